# baseline (speedup 1.0000x reference)
import math

import jax
import jax.numpy as jnp
from jax import lax
from jax.experimental import pallas as pl
from jax.experimental.pallas import tpu as pltpu
from jax.experimental.pallas import tpu_sc as plsc

_DIM = 512
_SCALE = math.sqrt(_DIM)
_NC, _NS, _L = 2, 16, 16
_NW = _NC * _NS
_CHUNK = 64


def _make_emb(B):
    b_per_w = B // _NW
    n_chunks = b_per_w // _CHUNK
    mesh = plsc.VectorSubcoreMesh(
        core_axis_name="c", subcore_axis_name="s",
        num_cores=_NC, num_subcores=_NS)

    def body(idx_hbm, table_hbm, out_hbm, idx_v, buf, sem):
        wid = lax.axis_index("s") * _NC + lax.axis_index("c")
        base = pl.multiple_of(wid * b_per_w, 8)
        pltpu.sync_copy(idx_hbm.at[pl.ds(base, b_per_w)], idx_v)

        def chunk_body(g, carry):
            off = pl.multiple_of(g * _CHUNK, 8)
            pltpu.async_copy(
                table_hbm.at[idx_v.at[pl.ds(off, _CHUNK)]], buf, sem).wait()

            def row_body(i, c):
                for j in range(_DIM // _L):
                    sl = pl.ds(j * _L, _L)
                    buf[i, sl] = buf[i, sl] * _SCALE
                return c

            lax.fori_loop(0, _CHUNK, row_body, 0)
            pltpu.sync_copy(buf, out_hbm.at[pl.ds(base + off, _CHUNK)])
            return carry

        lax.fori_loop(0, n_chunks, chunk_body, 0)

    return pl.kernel(
        body,
        out_type=jax.ShapeDtypeStruct((B, _DIM), jnp.float32),
        mesh=mesh,
        scratch_types=[
            pltpu.VMEM((b_per_w,), jnp.int32),
            pltpu.VMEM((_CHUNK, _DIM), jnp.float32),
            pltpu.SemaphoreType.DMA,
        ],
    )


def kernel(x, table):
    B = x.shape[0] * x.shape[1]
    idx = x.reshape(B)
    out = _make_emb(B)(idx, table)
    return out.reshape(x.shape[0], x.shape[1], _DIM)


# speedup vs baseline: 1.1723x; 1.1723x over previous
import math

import jax
import jax.numpy as jnp
from jax import lax
from jax.experimental import pallas as pl
from jax.experimental.pallas import tpu as pltpu
from jax.experimental.pallas import tpu_sc as plsc

_DIM = 512
_SCALE = math.sqrt(_DIM)
_NC, _NS, _L = 2, 16, 16
_NW = _NC * _NS
_CHUNK = 40


def _make_emb(B):
    b_per_w = B // _NW
    n_chunks = b_per_w // _CHUNK
    n_pairs = n_chunks // 2
    mesh = plsc.VectorSubcoreMesh(
        core_axis_name="c", subcore_axis_name="s",
        num_cores=_NC, num_subcores=_NS)

    def body(idx_hbm, table_hbm, out_hbm, idx_v,
             in0, in1, ob0, ob1, si0, si1, so0, so1):
        in_v = (in0, in1)
        out_v = (ob0, ob1)
        s_in = (si0, si1)
        s_out = (so0, so1)
        wid = lax.axis_index("s") * _NC + lax.axis_index("c")
        base = pl.multiple_of(wid * b_per_w, 8)
        pltpu.sync_copy(idx_hbm.at[pl.ds(base, b_per_w)], idx_v)

        def gather_start(c, b):
            off = pl.multiple_of(c * _CHUNK, 8)
            pltpu.async_copy(
                table_hbm.at[idx_v.at[pl.ds(off, _CHUNK)]], in_v[b], s_in[b])

        gather_start(0, 0)
        gather_start(1, 1)

        def pair_body(t, carry):
            for b in range(2):
                c = t * 2 + b
                pltpu.make_async_copy(
                    table_hbm.at[idx_v.at[pl.ds(0, _CHUNK)]],
                    in_v[b], s_in[b]).wait()
                @pl.when(t > 0)
                def _():
                    pltpu.make_async_copy(
                        out_v[b], out_hbm.at[pl.ds(0, _CHUNK)],
                        s_out[b]).wait()

                def row_body(i, cc):
                    for j in range(_DIM // _L):
                        sl = pl.ds(j * _L, _L)
                        out_v[b][i, sl] = in_v[b][i, sl] * _SCALE
                    return cc

                lax.fori_loop(0, _CHUNK, row_body, 0)
                off = pl.multiple_of(base + c * _CHUNK, 8)
                pltpu.async_copy(
                    out_v[b], out_hbm.at[pl.ds(off, _CHUNK)], s_out[b])

                @pl.when(c + 2 < n_chunks)
                def _():
                    gather_start(c + 2, b)
            return carry

        lax.fori_loop(0, n_pairs, pair_body, 0)
        for b in range(2):
            pltpu.make_async_copy(
                out_v[b], out_hbm.at[pl.ds(0, _CHUNK)], s_out[b]).wait()

    return pl.kernel(
        body,
        out_type=jax.ShapeDtypeStruct((B, _DIM), jnp.float32),
        mesh=mesh,
        scratch_types=[
            pltpu.VMEM((b_per_w,), jnp.int32),
            pltpu.VMEM((_CHUNK, _DIM), jnp.float32),
            pltpu.VMEM((_CHUNK, _DIM), jnp.float32),
            pltpu.VMEM((_CHUNK, _DIM), jnp.float32),
            pltpu.VMEM((_CHUNK, _DIM), jnp.float32),
            pltpu.SemaphoreType.DMA,
            pltpu.SemaphoreType.DMA,
            pltpu.SemaphoreType.DMA,
            pltpu.SemaphoreType.DMA,
        ],
    )


def kernel(x, table):
    B = x.shape[0] * x.shape[1]
    idx = x.reshape(B)
    out = _make_emb(B)(idx, table)
    return out.reshape(x.shape[0], x.shape[1], _DIM)


# speedup vs baseline: 1.3331x; 1.1372x over previous
import math

import jax
import jax.numpy as jnp
from jax import lax
from jax.experimental import pallas as pl
from jax.experimental.pallas import tpu as pltpu
from jax.experimental.pallas import tpu_sc as plsc

_DIM = 512
_SCALE = math.sqrt(_DIM)
_NC, _NS, _L = 2, 16, 16
_NW = _NC * _NS
_CHUNK = 40


_SEQ_PER_CHUNK = 2


def _make_emb(n_seq, seq_len):
    B = n_seq * seq_len
    b_per_w = B // _NW
    seq_per_w = n_seq // _NW
    n_chunks = seq_per_w // _SEQ_PER_CHUNK
    chunk_rows = _SEQ_PER_CHUNK * seq_len
    mesh = plsc.VectorSubcoreMesh(
        core_axis_name="c", subcore_axis_name="s",
        num_cores=_NC, num_subcores=_NS)

    def body(idx_hbm, table_hbm, out_hbm, idx_v,
             in0, in1, ob0, ob1, si0, si1, so0, so1):
        in_v = (in0, in1)
        out_v = (ob0, ob1)
        s_in = (si0, si1)
        s_out = (so0, so1)
        wid = lax.axis_index("s") * _NC + lax.axis_index("c")
        base = pl.multiple_of(wid * b_per_w, 8)
        seq_base = wid * seq_per_w
        pltpu.sync_copy(idx_hbm.at[pl.ds(base, b_per_w)], idx_v)

        def gather_start(c, b):
            off = pl.multiple_of(c * chunk_rows, 8)
            pltpu.async_copy(
                table_hbm.at[idx_v.at[pl.ds(off, chunk_rows)]],
                in_v[b], s_in[b])

        gather_start(0, 0)
        gather_start(1, 1)

        def pair_body(t, carry):
            for b in range(2):
                c = t * 2 + b
                pltpu.make_async_copy(
                    table_hbm.at[idx_v.at[pl.ds(0, chunk_rows)]],
                    in_v[b], s_in[b]).wait()
                @pl.when(t > 0)
                def _():
                    pltpu.make_async_copy(
                        out_v[b],
                        out_hbm.at[pl.ds(0, _SEQ_PER_CHUNK)],
                        s_out[b]).wait()

                def row_body(i, cc):
                    for a in range(_SEQ_PER_CHUNK):
                        for j in range(_DIM // _L):
                            sl = pl.ds(j * _L, _L)
                            out_v[b][a, i, sl] = (
                                in_v[b][a * seq_len + i, sl] * _SCALE)
                    return cc

                lax.fori_loop(0, seq_len, row_body, 0)
                soff = seq_base + c * _SEQ_PER_CHUNK
                pltpu.async_copy(
                    out_v[b], out_hbm.at[pl.ds(soff, _SEQ_PER_CHUNK)],
                    s_out[b])

                @pl.when(c + 2 < n_chunks)
                def _():
                    gather_start(c + 2, b)
            return carry

        lax.fori_loop(0, n_chunks // 2, pair_body, 0)
        for b in range(2):
            pltpu.make_async_copy(
                out_v[b], out_hbm.at[pl.ds(0, _SEQ_PER_CHUNK)],
                s_out[b]).wait()

    return pl.kernel(
        body,
        out_type=jax.ShapeDtypeStruct((n_seq, seq_len, _DIM), jnp.float32),
        mesh=mesh,
        scratch_types=[
            pltpu.VMEM((b_per_w,), jnp.int32),
            pltpu.VMEM((chunk_rows, _DIM), jnp.float32),
            pltpu.VMEM((chunk_rows, _DIM), jnp.float32),
            pltpu.VMEM((_SEQ_PER_CHUNK, seq_len, _DIM), jnp.float32),
            pltpu.VMEM((_SEQ_PER_CHUNK, seq_len, _DIM), jnp.float32),
            pltpu.SemaphoreType.DMA,
            pltpu.SemaphoreType.DMA,
            pltpu.SemaphoreType.DMA,
            pltpu.SemaphoreType.DMA,
        ],
    )


def kernel(x, table):
    n_seq, seq_len = x.shape
    idx = x.reshape(n_seq * seq_len)
    return _make_emb(n_seq, seq_len)(idx, table)


# speedup vs baseline: 1.3336x; 1.0004x over previous
import math

import jax
import jax.numpy as jnp
from jax import lax
from jax.experimental import pallas as pl
from jax.experimental.pallas import tpu as pltpu
from jax.experimental.pallas import tpu_sc as plsc

_DIM = 512
_SCALE = math.sqrt(_DIM)
_NC, _NS, _L = 2, 16, 16
_NW = _NC * _NS
_CHUNK = 40


_SEQ_PER_CHUNK = 2


def _make_emb(n_seq, seq_len):
    B = n_seq * seq_len
    b_per_w = B // _NW
    seq_per_w = n_seq // _NW
    n_chunks = seq_per_w // _SEQ_PER_CHUNK
    chunk_rows = _SEQ_PER_CHUNK * seq_len
    mesh = plsc.VectorSubcoreMesh(
        core_axis_name="c", subcore_axis_name="s",
        num_cores=_NC, num_subcores=_NS)

    def body(idx_hbm, table_hbm, out_hbm, idx_v,
             in0, in1, ob0, ob1, si0, si1, so0, so1):
        in_v = (in0, in1)
        out_v = (ob0, ob1)
        s_in = (si0, si1)
        s_out = (so0, so1)
        wid = lax.axis_index("s") * _NC + lax.axis_index("c")
        base = pl.multiple_of(wid * b_per_w, 8)
        seq_base = wid * seq_per_w
        pltpu.sync_copy(idx_hbm.at[pl.ds(base, b_per_w)], idx_v)

        def out_copies(b, soff, fn):
            for a in range(_SEQ_PER_CHUNK):
                fn(out_v[b].at[a, pl.ds(0, 16), :],
                   out_hbm.at[soff + a, pl.ds(0, 16), :], s_out[b])
                fn(out_v[b].at[a, pl.ds(16, seq_len - 16), :],
                   out_hbm.at[soff + a, pl.ds(16, seq_len - 16), :],
                   s_out[b])

        def gather_start(c, b):
            off = pl.multiple_of(c * chunk_rows, 8)
            pltpu.async_copy(
                table_hbm.at[idx_v.at[pl.ds(off, chunk_rows)]],
                in_v[b], s_in[b])

        gather_start(0, 0)
        gather_start(1, 1)

        def pair_body(t, carry):
            for b in range(2):
                c = t * 2 + b
                pltpu.make_async_copy(
                    table_hbm.at[idx_v.at[pl.ds(0, chunk_rows)]],
                    in_v[b], s_in[b]).wait()
                @pl.when(t > 0)
                def _():
                    out_copies(
                        b, 0,
                        lambda s, d, sem: pltpu.make_async_copy(
                            s, d, sem).wait())

                def row_body(i, cc):
                    for a in range(_SEQ_PER_CHUNK):
                        for j in range(_DIM // _L):
                            sl = pl.ds(j * _L, _L)
                            out_v[b][a, i, sl] = (
                                in_v[b][a * seq_len + i, sl] * _SCALE)
                    return cc

                lax.fori_loop(0, seq_len, row_body, 0)
                out_copies(b, seq_base + c * _SEQ_PER_CHUNK,
                           pltpu.async_copy)

                @pl.when(c + 2 < n_chunks)
                def _():
                    gather_start(c + 2, b)
            return carry

        lax.fori_loop(0, n_chunks // 2, pair_body, 0)
        for b in range(2):
            out_copies(
                b, 0,
                lambda s, d, sem: pltpu.make_async_copy(s, d, sem).wait())

    return pl.kernel(
        body,
        out_type=jax.ShapeDtypeStruct((n_seq, seq_len, _DIM), jnp.float32),
        mesh=mesh,
        scratch_types=[
            pltpu.VMEM((b_per_w,), jnp.int32),
            pltpu.VMEM((chunk_rows, _DIM), jnp.float32),
            pltpu.VMEM((chunk_rows, _DIM), jnp.float32),
            pltpu.VMEM((_SEQ_PER_CHUNK, seq_len, _DIM), jnp.float32),
            pltpu.VMEM((_SEQ_PER_CHUNK, seq_len, _DIM), jnp.float32),
            pltpu.SemaphoreType.DMA,
            pltpu.SemaphoreType.DMA,
            pltpu.SemaphoreType.DMA,
            pltpu.SemaphoreType.DMA,
        ],
    )


def kernel(x, table):
    n_seq, seq_len = x.shape
    idx = x.reshape(n_seq * seq_len)
    return _make_emb(n_seq, seq_len)(idx, table)
